# speedup vs baseline: 4.7829x; 4.7829x over previous
import functools

import jax
import jax.numpy as jnp
from jax import lax
from jax.experimental import pallas as pl
from jax.experimental.pallas import tpu as pltpu
from jax.experimental.pallas import tpu_sc as plsc

B, L, D = 16384, 20, 128
H, O = 1024, 1000

NC, NS = 2, 16
NW = NC * NS
ROWS_PER_W = B // NW
CB = 32
NCHUNK = ROWS_PER_W // CB
CROWS = CB * L
NLANE = D // 16


def _pool_body(xf_hbm, e_hbm, out_hbm, idx_v, rows_v, acc_v, sem):
    wid = lax.axis_index("s") * NC + lax.axis_index("c")

    def chunk(c, carry):
        row0 = wid * ROWS_PER_W + c * CB
        pltpu.sync_copy(xf_hbm.at[pl.ds(row0 * L, CROWS)], idx_v)
        pltpu.async_copy(e_hbm.at[idx_v], rows_v, sem).wait()

        def row(r, carry2):
            accs = [rows_v[r * L, pl.ds(d * 16, 16)] for d in range(NLANE)]
            for l in range(1, L):
                for d in range(NLANE):
                    accs[d] = accs[d] + rows_v[r * L + l, pl.ds(d * 16, 16)]
            for d in range(NLANE):
                acc_v[r, pl.ds(d * 16, 16)] = accs[d]
            return carry2

        lax.fori_loop(0, CB, row, 0)
        pltpu.sync_copy(acc_v, out_hbm.at[pl.ds(row0, CB)])
        return carry

    lax.fori_loop(0, NCHUNK, chunk, 0)


_pool = functools.partial(
    pl.kernel,
    out_type=jax.ShapeDtypeStruct((B, D), jnp.float32),
    mesh=plsc.VectorSubcoreMesh(core_axis_name="c", subcore_axis_name="s"),
    scratch_types=[
        pltpu.VMEM((CROWS,), jnp.int32),
        pltpu.VMEM((CROWS, D), jnp.float32),
        pltpu.VMEM((CB, D), jnp.float32),
        pltpu.SemaphoreType.DMA,
    ],
)(_pool_body)


def _mlp_body(x_ref, hs_ref, w1_ref, b1_ref, w2_ref, b2_ref, o_ref):
    cnt = jnp.sum((x_ref[...] != 0).astype(jnp.float32), axis=1, keepdims=True)
    h = hs_ref[...] / jnp.maximum(cnt, 1.0)
    h1 = jnp.dot(h, w1_ref[...], preferred_element_type=jnp.float32)
    h1 = jnp.maximum(h1 + b1_ref[...], 0.0)
    o_ref[...] = jnp.dot(h1, w2_ref[...],
                         preferred_element_type=jnp.float32) + b2_ref[...]


def _mlp(x, hsum, W1, b1, W2, b2):
    BM = 256
    return pl.pallas_call(
        _mlp_body,
        grid=(B // BM,),
        in_specs=[
            pl.BlockSpec((BM, L), lambda i: (i, 0)),
            pl.BlockSpec((BM, D), lambda i: (i, 0)),
            pl.BlockSpec((D, H), lambda i: (0, 0)),
            pl.BlockSpec((1, H), lambda i: (0, 0)),
            pl.BlockSpec((H, O), lambda i: (0, 0)),
            pl.BlockSpec((1, O), lambda i: (0, 0)),
        ],
        out_specs=pl.BlockSpec((BM, O), lambda i: (i, 0)),
        out_shape=jax.ShapeDtypeStruct((B, O), jnp.float32),
    )(x, hsum, W1, b1.reshape(1, H), W2, b2.reshape(1, O))


def kernel(x, E, W1, b1, W2, b2):
    x = x.astype(jnp.int32)
    hsum = _pool(x.reshape(-1), E)
    return _mlp(x, hsum, W1, b1, W2, b2)


# speedup vs baseline: 5.2933x; 1.1067x over previous
import functools

import jax
import jax.numpy as jnp
from jax import lax
from jax.experimental import pallas as pl
from jax.experimental.pallas import tpu as pltpu
from jax.experimental.pallas import tpu_sc as plsc

B, L, D = 16384, 20, 128
H, O = 1024, 1000

NCH = 4
BCH = B // NCH

NC, NS = 2, 16
NW = NC * NS
ROWS_PER_W = BCH // NW
CB = 32
NCHUNK = ROWS_PER_W // CB
CROWS = CB * L
NLANE = D // 16


def _pool_body(off, xf_hbm, e_hbm, out_hbm, idx_v, rows_v, acc_v, sem):
    wid = lax.axis_index("s") * NC + lax.axis_index("c")

    def chunk(c, carry):
        orow0 = wid * ROWS_PER_W + c * CB
        row0 = off + orow0
        pltpu.sync_copy(xf_hbm.at[pl.ds(row0 * L, CROWS)], idx_v)
        pltpu.async_copy(e_hbm.at[idx_v], rows_v, sem).wait()

        def row(r, carry2):
            accs = [rows_v[r * L, pl.ds(d * 16, 16)] for d in range(NLANE)]
            for l in range(1, L):
                for d in range(NLANE):
                    accs[d] = accs[d] + rows_v[r * L + l, pl.ds(d * 16, 16)]
            for d in range(NLANE):
                acc_v[r, pl.ds(d * 16, 16)] = accs[d]
            return carry2

        lax.fori_loop(0, CB, row, 0)
        pltpu.sync_copy(acc_v, out_hbm.at[pl.ds(orow0, CB)])
        return carry

    lax.fori_loop(0, NCHUNK, chunk, 0)


def _make_pool(ci):
    return functools.partial(
        pl.kernel,
        out_type=jax.ShapeDtypeStruct((BCH, D), jnp.float32),
        mesh=plsc.VectorSubcoreMesh(core_axis_name="c", subcore_axis_name="s"),
        scratch_types=[
            pltpu.VMEM((CROWS,), jnp.int32),
            pltpu.VMEM((CROWS, D), jnp.float32),
            pltpu.VMEM((CB, D), jnp.float32),
            pltpu.SemaphoreType.DMA,
        ],
    )(functools.partial(_pool_body, ci * BCH))


_POOLS = [_make_pool(ci) for ci in range(NCH)]

BM = 512
NBLK = BCH // BM


def _mlp_body(x_ref, hs_ref, w1_ref, b1_ref, w2_ref, b2_ref, o_ref):
    cnt = jnp.sum((x_ref[...] != 0).astype(jnp.float32), axis=1, keepdims=True)
    h = hs_ref[...] / jnp.maximum(cnt, 1.0)
    h1 = jnp.dot(h, w1_ref[...], preferred_element_type=jnp.float32)
    h1 = jnp.maximum(h1 + b1_ref[...], 0.0)
    o_ref[...] = jnp.dot(h1, w2_ref[...],
                         preferred_element_type=jnp.float32) + b2_ref[...]


def _mlp_chunk(ci, x, hs, W1, b1, W2, b2, acc):
    base = ci * NBLK
    in_specs = [
        pl.BlockSpec((BM, L), lambda i: (base + i, 0)),
        pl.BlockSpec((BM, D), lambda i: (i, 0)),
        pl.BlockSpec((D, H), lambda i: (0, 0)),
        pl.BlockSpec((1, H), lambda i: (0, 0)),
        pl.BlockSpec((H, O), lambda i: (0, 0)),
        pl.BlockSpec((1, O), lambda i: (0, 0)),
    ]
    args = [x, hs, W1, b1.reshape(1, H), W2, b2.reshape(1, O)]
    kwargs = {}
    body = _mlp_body
    if acc is not None:
        in_specs.append(pl.BlockSpec(memory_space=pl.ANY))
        args.append(acc)
        kwargs["input_output_aliases"] = {6: 0}
        body = lambda x_r, hs_r, w1_r, b1_r, w2_r, b2_r, a_r, o_r: (
            _mlp_body(x_r, hs_r, w1_r, b1_r, w2_r, b2_r, o_r))
    return pl.pallas_call(
        body,
        grid=(NBLK,),
        in_specs=in_specs,
        out_specs=pl.BlockSpec((BM, O), lambda i: (base + i, 0)),
        out_shape=jax.ShapeDtypeStruct((B, O), jnp.float32),
        **kwargs,
    )(*args)


def kernel(x, E, W1, b1, W2, b2):
    x = x.astype(jnp.int32)
    xf = x.reshape(-1)
    hs = [_POOLS[ci](xf, E) for ci in range(NCH)]
    out = None
    for ci in range(NCH):
        out = _mlp_chunk(ci, x, hs[ci], W1, b1, W2, b2, out)
    return out


# speedup vs baseline: 6.7385x; 1.2730x over previous
import functools

import jax
import jax.numpy as jnp
from jax import lax
from jax.experimental import pallas as pl
from jax.experimental.pallas import tpu as pltpu
from jax.experimental.pallas import tpu_sc as plsc

B, L, D = 16384, 20, 128
H, O = 1024, 1000

NCH = 4
BCH = B // NCH

NC, NS = 2, 16
NW = NC * NS
ROWS_PER_W = BCH // NW
CB = 32
NCHUNK = ROWS_PER_W // CB
CROWS = CB * L
NLANE = D // 16


def _pool_body(off, xf_hbm, e_hbm, out_hbm, idx_v, rows_v, acc_v, sem):
    wid = lax.axis_index("s") * NC + lax.axis_index("c")

    def chunk(c, carry):
        orow0 = wid * ROWS_PER_W + c * CB
        row0 = off + orow0
        pltpu.sync_copy(xf_hbm.at[pl.ds(row0 * L, CROWS)], idx_v)
        pltpu.async_copy(e_hbm.at[idx_v], rows_v, sem).wait()

        def row(r, carry2):
            accs = [rows_v[r * L, pl.ds(d * 16, 16)] for d in range(NLANE)]
            for l in range(1, L):
                for d in range(NLANE):
                    accs[d] = accs[d] + rows_v[r * L + l, pl.ds(d * 16, 16)]
            for d in range(NLANE):
                acc_v[r, pl.ds(d * 16, 16)] = accs[d]
            return carry2

        lax.fori_loop(0, CB, row, 0)
        pltpu.sync_copy(acc_v, out_hbm.at[pl.ds(orow0, CB)])
        return carry

    lax.fori_loop(0, NCHUNK, chunk, 0)


def _make_pool(ci):
    return functools.partial(
        pl.kernel,
        out_type=jax.ShapeDtypeStruct((BCH, D), jnp.float32),
        mesh=plsc.VectorSubcoreMesh(core_axis_name="c", subcore_axis_name="s"),
        scratch_types=[
            pltpu.VMEM((CROWS,), jnp.int32),
            pltpu.VMEM((CROWS, D), jnp.float32),
            pltpu.VMEM((CB, D), jnp.float32),
            pltpu.SemaphoreType.DMA,
        ],
    )(functools.partial(_pool_body, ci * BCH))


_POOLS = [_make_pool(ci) for ci in range(NCH)]

BM = 512
NBLK = BCH // BM


def _mlp_body(x_ref, hs_ref, w1t_ref, b1_ref, w2t_ref, b2_ref, o_ref):
    cnt = jnp.sum((x_ref[...] != 0).astype(jnp.float32), axis=1, keepdims=True)
    h = hs_ref[...] / jnp.maximum(cnt, 1.0)
    h1t = lax.dot_general(w1t_ref[...], h, (((1,), (1,)), ((), ())),
                          preferred_element_type=jnp.float32)
    h1t = jnp.maximum(h1t + b1_ref[...], 0.0)
    o_ref[...] = jnp.dot(w2t_ref[...], h1t,
                         preferred_element_type=jnp.float32) + b2_ref[...]


def _mlp_chunk(ci, x, hs, W1t, b1c, W2t, b2c, acc):
    base = ci * NBLK
    in_specs = [
        pl.BlockSpec((BM, L), lambda i: (base + i, 0)),
        pl.BlockSpec((BM, D), lambda i: (i, 0)),
        pl.BlockSpec((H, D), lambda i: (0, 0)),
        pl.BlockSpec((H, 1), lambda i: (0, 0)),
        pl.BlockSpec((O, H), lambda i: (0, 0)),
        pl.BlockSpec((O, 1), lambda i: (0, 0)),
    ]
    args = [x, hs, W1t, b1c, W2t, b2c]
    kwargs = {}
    body = _mlp_body
    if acc is not None:
        in_specs.append(pl.BlockSpec(memory_space=pl.ANY))
        args.append(acc)
        kwargs["input_output_aliases"] = {6: 0}
        body = lambda x_r, hs_r, w1_r, b1_r, w2_r, b2_r, a_r, o_r: (
            _mlp_body(x_r, hs_r, w1_r, b1_r, w2_r, b2_r, o_r))
    return pl.pallas_call(
        body,
        grid=(NBLK,),
        in_specs=in_specs,
        out_specs=pl.BlockSpec((O, BM), lambda i: (0, base + i)),
        out_shape=jax.ShapeDtypeStruct((O, B), jnp.float32),
        **kwargs,
    )(*args)


def kernel(x, E, W1, b1, W2, b2):
    x = x.astype(jnp.int32)
    xf = x.reshape(-1)
    W1t = W1.T
    W2t = W2.T
    b1c = b1.reshape(H, 1)
    b2c = b2.reshape(O, 1)
    hs = [_POOLS[ci](xf, E) for ci in range(NCH)]
    outt = None
    for ci in range(NCH):
        outt = _mlp_chunk(ci, x, hs[ci], W1t, b1c, W2t, b2c, outt)
    return outt.T


# speedup vs baseline: 7.9029x; 1.1728x over previous
import functools

import jax
import jax.numpy as jnp
from jax import lax
from jax.experimental import pallas as pl
from jax.experimental.pallas import tpu as pltpu
from jax.experimental.pallas import tpu_sc as plsc

B, L, D = 16384, 20, 128
H, O = 1024, 1000

NCH = 4
BCH = B // NCH

NC, NS = 2, 16
NW = NC * NS
ROWS_PER_W = BCH // NW
CB = 16
NCHUNK = ROWS_PER_W // CB
CROWS = CB * L
NLANE = D // 16


def _pool_body(off, xf_hbm, e_hbm, out_hbm,
               idx0, idx1, rows0, rows1, acc_v, sem0, sem1):
    wid = lax.axis_index("s") * NC + lax.axis_index("c")
    base = wid * ROWS_PER_W
    idx = (idx0, idx1)
    rows = (rows0, rows1)
    sem = (sem0, sem1)

    def stage(c):
        b = c % 2
        pltpu.sync_copy(xf_hbm.at[pl.ds((off + base + c * CB) * L, CROWS)],
                        idx[b])
        pltpu.async_copy(e_hbm.at[idx[b]], rows[b], sem[b])

    def accum(c):
        b = c % 2
        pltpu.make_async_copy(e_hbm.at[idx[b]], rows[b], sem[b]).wait()
        rv = rows[b]

        def row(r, carry):
            accs = [rv[r * L, pl.ds(d * 16, 16)] for d in range(NLANE)]
            for l in range(1, L):
                for d in range(NLANE):
                    accs[d] = accs[d] + rv[r * L + l, pl.ds(d * 16, 16)]
            for d in range(NLANE):
                acc_v[r, pl.ds(d * 16, 16)] = accs[d]
            return carry

        lax.fori_loop(0, CB, row, 0)
        pltpu.sync_copy(acc_v, out_hbm.at[pl.ds(base + c * CB, CB)])

    stage(0)
    for c in range(NCHUNK):
        if c + 1 < NCHUNK:
            stage(c + 1)
        accum(c)


def _make_pool(ci):
    return functools.partial(
        pl.kernel,
        out_type=jax.ShapeDtypeStruct((BCH, D), jnp.float32),
        mesh=plsc.VectorSubcoreMesh(core_axis_name="c", subcore_axis_name="s"),
        scratch_types=[
            pltpu.VMEM((CROWS,), jnp.int32),
            pltpu.VMEM((CROWS,), jnp.int32),
            pltpu.VMEM((CROWS, D), jnp.float32),
            pltpu.VMEM((CROWS, D), jnp.float32),
            pltpu.VMEM((CB, D), jnp.float32),
            pltpu.SemaphoreType.DMA,
            pltpu.SemaphoreType.DMA,
        ],
    )(functools.partial(_pool_body, ci * BCH))


_POOLS = [_make_pool(ci) for ci in range(NCH)]

BM = 512
NBLK = BCH // BM


def _mlp_body(x_ref, hs_ref, w1t_ref, b1_ref, w2t_ref, b2_ref, o_ref):
    cnt = jnp.sum((x_ref[...] != 0).astype(jnp.float32), axis=1, keepdims=True)
    h = hs_ref[...] / jnp.maximum(cnt, 1.0)
    h1t = lax.dot_general(w1t_ref[...], h, (((1,), (1,)), ((), ())),
                          preferred_element_type=jnp.float32)
    h1t = jnp.maximum(h1t + b1_ref[...], 0.0)
    o_ref[...] = jnp.dot(w2t_ref[...], h1t,
                         preferred_element_type=jnp.float32) + b2_ref[...]


def _mlp_chunk(ci, x, hs, W1t, b1c, W2t, b2c, acc):
    base = ci * NBLK
    in_specs = [
        pl.BlockSpec((BM, L), lambda i: (base + i, 0)),
        pl.BlockSpec((BM, D), lambda i: (i, 0)),
        pl.BlockSpec((H, D), lambda i: (0, 0)),
        pl.BlockSpec((H, 1), lambda i: (0, 0)),
        pl.BlockSpec((O, H), lambda i: (0, 0)),
        pl.BlockSpec((O, 1), lambda i: (0, 0)),
    ]
    args = [x, hs, W1t, b1c, W2t, b2c]
    kwargs = {}
    body = _mlp_body
    if acc is not None:
        in_specs.append(pl.BlockSpec(memory_space=pl.ANY))
        args.append(acc)
        kwargs["input_output_aliases"] = {6: 0}
        body = lambda x_r, hs_r, w1_r, b1_r, w2_r, b2_r, a_r, o_r: (
            _mlp_body(x_r, hs_r, w1_r, b1_r, w2_r, b2_r, o_r))
    return pl.pallas_call(
        body,
        grid=(NBLK,),
        in_specs=in_specs,
        out_specs=pl.BlockSpec((O, BM), lambda i: (0, base + i)),
        out_shape=jax.ShapeDtypeStruct((O, B), jnp.float32),
        **kwargs,
    )(*args)


def kernel(x, E, W1, b1, W2, b2):
    x = x.astype(jnp.int32)
    xf = x.reshape(-1)
    W1t = W1.T
    W2t = W2.T
    b1c = b1.reshape(H, 1)
    b2c = b2.reshape(O, 1)
    hs = [_POOLS[ci](xf, E) for ci in range(NCH)]
    outt = None
    for ci in range(NCH):
        outt = _mlp_chunk(ci, x, hs[ci], W1t, b1c, W2t, b2c, outt)
    return outt.T
